# baseline (speedup 1.0000x reference)
import functools

import jax
import jax.numpy as jnp
from jax import lax
from jax.experimental import pallas as pl
from jax.experimental.pallas import tpu as pltpu
from jax.experimental.pallas import tpu_sc as plsc

_S = 10000
_NW = 32
_B = 200
_G = 8
_ACC_W = 640
_ZB = 8
_K = 8

_NEG = float("-inf")
_POS = float("inf")


def _sc_segment_acc(feats, ids):
    n, d = feats.shape
    nj = d // 16
    chunk = n // _NW
    mesh = plsc.VectorSubcoreMesh(core_axis_name="c", subcore_axis_name="s")

    @functools.partial(
        pl.kernel,
        out_type=jax.ShapeDtypeStruct((_S * _ACC_W,), jnp.float32),
        mesh=mesh,
        scratch_types=[
            pltpu.VMEM((_B + 16,), jnp.int32),
            pltpu.VMEM((_B + 16,), jnp.int32),
            pltpu.VMEM((_B, d), jnp.float32),
            pltpu.VMEM((_B, d), jnp.float32),
            pltpu.VMEM((_ACC_W,), jnp.float32),
            pltpu.VMEM((_K, _ACC_W), jnp.float32),
            pltpu.VMEM((_ZB * _ACC_W,), jnp.float32),
            pltpu.VMEM((16,), jnp.int32),
            pltpu.SMEM((8,), jnp.int32),
            pltpu.SemaphoreType.DMA,
            pltpu.SemaphoreType.DMA,
            pltpu.SemaphoreType.DMA,
        ],
    )
    def sc_kernel(feats_hbm, ids_hbm, acc_hbm, ids0, ids1, fv0, fv1,
                  accv, stg, zv, bv, st_s, fsem0, fsem1, wsem):
        w = lax.axis_index("c") * 16 + lax.axis_index("s")
        cs = w * chunk
        ce = cs + chunk

        def issue_fetch(r, idsb, fvb, fsem):
            pltpu.async_copy(
                ids_hbm.at[pl.ds(pl.multiple_of(r, 8), _B)],
                idsb.at[pl.ds(0, _B)], fsem)
            pltpu.async_copy(
                feats_hbm.at[pl.ds(pl.multiple_of(r, 8), _B)], fvb, fsem)

        def wait_fetch(idsb, fvb, fsem):
            pltpu.make_async_copy(
                ids_hbm.at[pl.ds(0, _B)], idsb.at[pl.ds(0, _B)], fsem).wait()
            pltpu.make_async_copy(
                feats_hbm.at[pl.ds(0, _B)], fvb, fsem).wait()

        issue_fetch(cs, ids0, fv0, fsem0)
        st_s[5] = jnp.int32(1)
        st_s[6] = jnp.int32(0)

        @pl.loop(0, _ZB * _ACC_W, step=16)
        def _(i):
            zv[pl.ds(i, 16)] = jnp.zeros((16,), jnp.float32)

        @pl.loop(0, _ACC_W, step=16)
        def _(i):
            accv[pl.ds(i, 16)] = jnp.zeros((16,), jnp.float32)

        pltpu.sync_copy(
            ids_hbm.at[pl.ds(pl.multiple_of(jnp.maximum(cs - 16, 0), 8), 16)],
            bv)
        prev = jnp.where(w > 0, bv[pl.ds(0, 16)][15], -1)
        pltpu.sync_copy(ids_hbm.at[pl.ds(pl.multiple_of(ce - 16, 8), 16)], bv)
        hi = jnp.where(w < _NW - 1, bv[pl.ds(0, 16)][15] + 1, _S)
        lo = prev + 1

        lo8 = ((lo + _ZB - 1) // _ZB) * _ZB
        hi8 = (hi // _ZB) * _ZB

        def zero_row(z):
            pltpu.sync_copy(
                zv.at[pl.ds(0, _ACC_W)],
                acc_hbm.at[pl.ds(pl.multiple_of(z * _ACC_W, 8), _ACC_W)])

        for t in range(_ZB - 1):
            @pl.when(lo + t < jnp.minimum(hi, lo8))
            def _(t=t):
                zero_row(lo + t)

        for t in range(_ZB - 1):
            @pl.when((hi8 + t >= lo) & (hi8 + t < hi))
            def _(t=t):
                zero_row(hi8 + t)

        @pl.loop(0, _S // _ZB)
        def _(b):
            z = b * _ZB

            @pl.when((z >= lo8) & (z < hi8))
            def _():
                pltpu.sync_copy(
                    zv,
                    acc_hbm.at[pl.ds(pl.multiple_of(z * _ACC_W, 8),
                                     _ZB * _ACC_W)])

        def drain_one():
            pltpu.make_async_copy(
                stg.at[0], acc_hbm.at[pl.ds(0, _ACC_W)], wsem).wait()

        def fire_acc(cur, cnt):
            m = st_s[4]
            sl = lax.rem(m, _K)
            accv[pl.ds(512, 16)] = jnp.full((16,), cnt.astype(jnp.float32))
            for q in range(_ACC_W // 16):
                stg[sl, pl.ds(q * 16, 16)] = accv[pl.ds(q * 16, 16)]
            pltpu.async_copy(
                stg.at[sl],
                acc_hbm.at[pl.ds(pl.multiple_of(cur * _ACC_W, 8), _ACC_W)],
                wsem)
            st_s[4] = m + 1

            @pl.when(lax.rem(m + 1, _K) == 0)
            def _():
                for _q in range(_K):
                    drain_one()

        st_s[0] = jnp.int32(-1)
        st_s[1] = jnp.int32(0)
        st_s[3] = jnp.int32(0)
        st_s[4] = jnp.int32(0)

        def process_block(r0, idsb, fvb):
            @pl.loop(0, _B)
            def _(i):
                sid = idsb[pl.ds(i, 16)][0]
                cur = st_s[0]
                cnt = st_s[1]
                live = st_s[3] == 0
                valid = live & (sid < hi) & (sid >= lo)
                is_new = valid & (sid != cur)

                @pl.when(live & (sid >= hi))
                def _():
                    st_s[3] = jnp.int32(1)

                @pl.when(is_new)
                def _():
                    @pl.when(cnt > 0)
                    def _():
                        fire_acc(cur, cnt)

                    for j in range(nj):
                        x = fvb[i, pl.ds(j * 16, 16)]
                        accv[pl.ds(j * 16, 16)] = x
                        accv[pl.ds(128 + j * 16, 16)] = x * x
                        accv[pl.ds(256 + j * 16, 16)] = x
                        accv[pl.ds(384 + j * 16, 16)] = x
                    st_s[0] = sid
                    st_s[1] = jnp.int32(1)

                @pl.when(valid & jnp.logical_not(is_new))
                def _():
                    for j in range(nj):
                        x = fvb[i, pl.ds(j * 16, 16)]
                        accv[pl.ds(j * 16, 16)] += x
                        accv[pl.ds(128 + j * 16, 16)] += x * x
                        accv[pl.ds(256 + j * 16, 16)] = jnp.maximum(
                            accv[pl.ds(256 + j * 16, 16)], x)
                        accv[pl.ds(384 + j * 16, 16)] = jnp.minimum(
                            accv[pl.ds(384 + j * 16, 16)], x)
                    st_s[1] = cnt + 1

        def process_block_grouped(r0, idsb, fvb):
            @pl.loop(0, _B // _G)
            def _(g):
                t0 = g * _G
                idv = idsb[pl.ds(t0, 16)]
                f = idv[0]
                last = idv[_G - 1]
                cur = st_s[0]
                cnt = st_s[1]
                live = st_s[3] == 0
                uni = f == last

                @pl.when(live & uni & (f >= hi))
                def _():
                    st_s[3] = jnp.int32(1)

                fast = live & uni & (f >= lo) & (f < hi)
                ext = f == cur

                def group_stats(j):
                    jo = j * 16
                    x = [fvb[t0 + t, pl.ds(jo, 16)] for t in range(_G)]
                    s = (((x[0] + x[1]) + (x[2] + x[3]))
                         + ((x[4] + x[5]) + (x[6] + x[7])))
                    q = (((x[0] * x[0] + x[1] * x[1])
                          + (x[2] * x[2] + x[3] * x[3]))
                         + ((x[4] * x[4] + x[5] * x[5])
                            + (x[6] * x[6] + x[7] * x[7])))
                    mx = jnp.maximum(
                        jnp.maximum(jnp.maximum(x[0], x[1]),
                                    jnp.maximum(x[2], x[3])),
                        jnp.maximum(jnp.maximum(x[4], x[5]),
                                    jnp.maximum(x[6], x[7])))
                    mn = jnp.minimum(
                        jnp.minimum(jnp.minimum(x[0], x[1]),
                                    jnp.minimum(x[2], x[3])),
                        jnp.minimum(jnp.minimum(x[4], x[5]),
                                    jnp.minimum(x[6], x[7])))
                    return jo, s, q, mx, mn

                @pl.when(fast & ext)
                def _():
                    for j in range(nj):
                        jo, s, q, mx, mn = group_stats(j)
                        accv[pl.ds(jo, 16)] += s
                        accv[pl.ds(128 + jo, 16)] += q
                        accv[pl.ds(256 + jo, 16)] = jnp.maximum(
                            accv[pl.ds(256 + jo, 16)], mx)
                        accv[pl.ds(384 + jo, 16)] = jnp.minimum(
                            accv[pl.ds(384 + jo, 16)], mn)
                    st_s[1] = cnt + _G

                @pl.when(fast & jnp.logical_not(ext))
                def _():
                    @pl.when(cnt > 0)
                    def _():
                        fire_acc(cur, cnt)

                    for j in range(nj):
                        jo, s, q, mx, mn = group_stats(j)
                        accv[pl.ds(jo, 16)] = s
                        accv[pl.ds(128 + jo, 16)] = q
                        accv[pl.ds(256 + jo, 16)] = mx
                        accv[pl.ds(384 + jo, 16)] = mn
                    st_s[0] = f
                    st_s[1] = jnp.int32(_G)

                @pl.when(live & jnp.logical_not(uni))
                def _():
                    for t in range(_G):
                        sid = idsb[pl.ds(t0 + t, 16)][0]
                        cur_t = st_s[0]
                        cnt_t = st_s[1]
                        live_t = st_s[3] == 0
                        valid = live_t & (sid < hi) & (sid >= lo)
                        is_new = valid & (sid != cur_t)

                        @pl.when(live_t & (sid >= hi))
                        def _():
                            st_s[3] = jnp.int32(1)

                        @pl.when(is_new)
                        def _(t=t, cur_t=cur_t, cnt_t=cnt_t):
                            @pl.when(cnt_t > 0)
                            def _():
                                fire_acc(cur_t, cnt_t)

                            for j in range(nj):
                                xx = fvb[t0 + t, pl.ds(j * 16, 16)]
                                accv[pl.ds(j * 16, 16)] = xx
                                accv[pl.ds(128 + j * 16, 16)] = xx * xx
                                accv[pl.ds(256 + j * 16, 16)] = xx
                                accv[pl.ds(384 + j * 16, 16)] = xx
                            st_s[0] = sid
                            st_s[1] = jnp.int32(1)

                        @pl.when(valid & jnp.logical_not(is_new))
                        def _(t=t, cnt_t=cnt_t):
                            for j in range(nj):
                                xx = fvb[t0 + t, pl.ds(j * 16, 16)]
                                accv[pl.ds(j * 16, 16)] += xx
                                accv[pl.ds(128 + j * 16, 16)] += xx * xx
                                accv[pl.ds(256 + j * 16, 16)] = jnp.maximum(
                                    accv[pl.ds(256 + j * 16, 16)], xx)
                                accv[pl.ds(384 + j * 16, 16)] = jnp.minimum(
                                    accv[pl.ds(384 + j * 16, 16)], xx)
                            st_s[1] = cnt_t + 1

        @pl.loop(0, n // (2 * _B))
        def _(kk):
            r0 = cs + (2 * kk) * _B
            r1 = r0 + _B

            @pl.when((st_s[3] == 0) & (r0 < n))
            def _():
                @pl.when(r1 < n)
                def _():
                    issue_fetch(r1, ids1, fv1, fsem1)
                    st_s[6] = jnp.int32(1)

                wait_fetch(ids0, fv0, fsem0)
                st_s[5] = jnp.int32(0)
                process_block_grouped(r0, ids0, fv0)

            @pl.when((st_s[3] == 0) & (r1 < n))
            def _():
                @pl.when(r1 + _B < n)
                def _():
                    issue_fetch(r1 + _B, ids0, fv0, fsem0)
                    st_s[5] = jnp.int32(1)

                wait_fetch(ids1, fv1, fsem1)
                st_s[6] = jnp.int32(0)
                process_block_grouped(r1, ids1, fv1)

        @pl.when(st_s[1] > 0)
        def _():
            fire_acc(st_s[0], st_s[1])

        @pl.when(st_s[5] == 1)
        def _():
            wait_fetch(ids0, fv0, fsem0)

        @pl.when(st_s[6] == 1)
        def _():
            wait_fetch(ids1, fv1, fsem1)

        for t in range(_K):
            @pl.when(t < lax.rem(st_s[4], _K))
            def _():
                drain_one()

    return sc_kernel(feats, ids)


def _tc_finalize(acc):
    bs = 400

    def body(acc_ref, out_ref):
        a = acc_ref[...]
        sm = a[:, 0:128]
        sq = a[:, 128:256]
        mx = a[:, 256:384]
        mn = a[:, 384:512]
        cnt = a[:, 512:513]
        c1 = jnp.maximum(cnt, 1.0)
        mean = sm / c1
        var = (sq - cnt * mean * mean) / jnp.maximum(cnt - 1.0, 1.0)
        std = jnp.sqrt(jnp.clip(var, 0.0) + 1e-12)
        pos = cnt > 0.0
        out_ref[:, 0:128] = mean
        out_ref[:, 128:256] = std
        out_ref[:, 256:384] = jnp.where(pos, mx, 0.0)
        out_ref[:, 384:512] = jnp.where(pos, mn, 0.0)

    return pl.pallas_call(
        body,
        grid=(_S // bs,),
        in_specs=[pl.BlockSpec((bs, _ACC_W), lambda i: (i, 0))],
        out_specs=pl.BlockSpec((bs, 512), lambda i: (i, 0)),
        out_shape=jax.ShapeDtypeStruct((_S, 512), jnp.float32),
    )(acc)


def kernel(feats, segment_ids):
    ids = segment_ids.astype(jnp.int32)
    acc = _sc_segment_acc(feats, ids)
    return _tc_finalize(acc.reshape(_S, _ACC_W))


# speedup vs baseline: 3.2684x; 3.2684x over previous
import functools

import jax
import jax.numpy as jnp
from jax import lax
from jax.experimental import pallas as pl
from jax.experimental.pallas import tpu as pltpu
from jax.experimental.pallas import tpu_sc as plsc

_S = 10000
_NW = 32
_B = 200
_G = 8
_ACC_W = 640
_ZB = 8
_K = 8

_NEG = float("-inf")
_POS = float("inf")


def _sc_segment_acc(feats, ids):
    n, d = feats.shape
    nj = d // 16
    chunk = n // _NW
    mesh = plsc.VectorSubcoreMesh(core_axis_name="c", subcore_axis_name="s")

    @functools.partial(
        pl.kernel,
        out_type=jax.ShapeDtypeStruct((_S * _ACC_W,), jnp.float32),
        mesh=mesh,
        scratch_types=[
            pltpu.VMEM((_B + 16,), jnp.int32),
            pltpu.VMEM((_B + 16,), jnp.int32),
            pltpu.VMEM((_B, d), jnp.float32),
            pltpu.VMEM((_B, d), jnp.float32),
            pltpu.VMEM((_ACC_W,), jnp.float32),
            pltpu.VMEM((_K, _ACC_W), jnp.float32),
            pltpu.VMEM((_ZB * _ACC_W,), jnp.float32),
            pltpu.VMEM((16,), jnp.int32),
            pltpu.SMEM((8,), jnp.int32),
            pltpu.SemaphoreType.DMA,
            pltpu.SemaphoreType.DMA,
            pltpu.SemaphoreType.DMA,
        ],
    )
    def sc_kernel(feats_hbm, ids_hbm, acc_hbm, ids0, ids1, fv0, fv1,
                  accv, stg, zv, bv, st_s, fsem0, fsem1, wsem):
        w = lax.axis_index("c") * 16 + lax.axis_index("s")
        cs = w * chunk
        ce = cs + chunk

        def issue_fetch(r, idsb, fvb, fsem):
            pltpu.async_copy(
                ids_hbm.at[pl.ds(pl.multiple_of(r, 8), _B)],
                idsb.at[pl.ds(0, _B)], fsem)
            pltpu.async_copy(
                feats_hbm.at[pl.ds(pl.multiple_of(r, 8), _B)], fvb, fsem)

        def wait_fetch(idsb, fvb, fsem):
            pltpu.make_async_copy(
                ids_hbm.at[pl.ds(0, _B)], idsb.at[pl.ds(0, _B)], fsem).wait()
            pltpu.make_async_copy(
                feats_hbm.at[pl.ds(0, _B)], fvb, fsem).wait()

        issue_fetch(cs, ids0, fv0, fsem0)
        st_s[5] = jnp.int32(1)
        st_s[6] = jnp.int32(0)

        @pl.loop(0, _ZB * _ACC_W, step=16)
        def _(i):
            zv[pl.ds(i, 16)] = jnp.zeros((16,), jnp.float32)

        @pl.loop(0, _ACC_W, step=16)
        def _(i):
            accv[pl.ds(i, 16)] = jnp.zeros((16,), jnp.float32)

        pltpu.sync_copy(
            ids_hbm.at[pl.ds(pl.multiple_of(jnp.maximum(cs - 16, 0), 8), 16)],
            bv)
        prev = jnp.where(w > 0, bv[pl.ds(0, 16)][15], -1)
        pltpu.sync_copy(ids_hbm.at[pl.ds(pl.multiple_of(ce - 16, 8), 16)], bv)
        hi = jnp.where(w < _NW - 1, bv[pl.ds(0, 16)][15] + 1, _S)
        lo = prev + 1

        lo8 = ((lo + _ZB - 1) // _ZB) * _ZB
        hi8 = (hi // _ZB) * _ZB

        def zero_row(z):
            pltpu.sync_copy(
                zv.at[pl.ds(0, _ACC_W)],
                acc_hbm.at[pl.ds(pl.multiple_of(z * _ACC_W, 8), _ACC_W)])

        for t in range(_ZB - 1):
            @pl.when(lo + t < jnp.minimum(hi, lo8))
            def _(t=t):
                zero_row(lo + t)

        for t in range(_ZB - 1):
            @pl.when((hi8 + t >= lo) & (hi8 + t < hi))
            def _(t=t):
                zero_row(hi8 + t)

        @pl.loop(0, _S // _ZB)
        def _(b):
            z = b * _ZB

            @pl.when((z >= lo8) & (z < hi8))
            def _():
                pltpu.sync_copy(
                    zv,
                    acc_hbm.at[pl.ds(pl.multiple_of(z * _ACC_W, 8),
                                     _ZB * _ACC_W)])

        def drain_one():
            pltpu.make_async_copy(
                stg.at[0], acc_hbm.at[pl.ds(0, _ACC_W)], wsem).wait()

        def fire_acc(cur, cnt):
            m = st_s[4]
            sl = lax.rem(m, _K)
            accv[pl.ds(512, 16)] = jnp.full((16,), cnt.astype(jnp.float32))
            for q in range(_ACC_W // 16):
                stg[sl, pl.ds(q * 16, 16)] = accv[pl.ds(q * 16, 16)]
            pltpu.async_copy(
                stg.at[sl],
                acc_hbm.at[pl.ds(pl.multiple_of(cur * _ACC_W, 8), _ACC_W)],
                wsem)
            st_s[4] = m + 1

            @pl.when(lax.rem(m + 1, _K) == 0)
            def _():
                for _q in range(_K):
                    drain_one()

        st_s[0] = jnp.int32(-1)
        st_s[1] = jnp.int32(0)
        st_s[3] = jnp.int32(0)
        st_s[4] = jnp.int32(0)

        def process_block(r0, idsb, fvb):
            @pl.loop(0, _B)
            def _(i):
                sid = idsb[pl.ds(i, 16)][0]
                cur = st_s[0]
                cnt = st_s[1]
                live = st_s[3] == 0
                valid = live & (sid < hi) & (sid >= lo)
                is_new = valid & (sid != cur)

                @pl.when(live & (sid >= hi))
                def _():
                    st_s[3] = jnp.int32(1)

                @pl.when(is_new)
                def _():
                    @pl.when(cnt > 0)
                    def _():
                        fire_acc(cur, cnt)

                    for j in range(nj):
                        x = fvb[i, pl.ds(j * 16, 16)]
                        accv[pl.ds(j * 16, 16)] = x
                        accv[pl.ds(128 + j * 16, 16)] = x * x
                        accv[pl.ds(256 + j * 16, 16)] = x
                        accv[pl.ds(384 + j * 16, 16)] = x
                    st_s[0] = sid
                    st_s[1] = jnp.int32(1)

                @pl.when(valid & jnp.logical_not(is_new))
                def _():
                    for j in range(nj):
                        x = fvb[i, pl.ds(j * 16, 16)]
                        accv[pl.ds(j * 16, 16)] += x
                        accv[pl.ds(128 + j * 16, 16)] += x * x
                        accv[pl.ds(256 + j * 16, 16)] = jnp.maximum(
                            accv[pl.ds(256 + j * 16, 16)], x)
                        accv[pl.ds(384 + j * 16, 16)] = jnp.minimum(
                            accv[pl.ds(384 + j * 16, 16)], x)
                    st_s[1] = cnt + 1

        def process_block_grouped(r0, idsb, fvb):
            @pl.loop(0, _B // _G)
            def _(g):
                t0 = g * _G
                idv = idsb[pl.ds(t0, 16)]
                f = idv[0]
                last = idv[_G - 1]
                cur = st_s[0]
                cnt = st_s[1]
                live = st_s[3] == 0
                uni = f == last

                @pl.when(live & uni & (f >= hi))
                def _():
                    st_s[3] = jnp.int32(1)

                fast = live & uni & (f >= lo) & (f < hi)
                ext = f == cur

                def group_stats(j):
                    jo = j * 16
                    x = [fvb[t0 + t, pl.ds(jo, 16)] for t in range(_G)]
                    s = (((x[0] + x[1]) + (x[2] + x[3]))
                         + ((x[4] + x[5]) + (x[6] + x[7])))
                    q = (((x[0] * x[0] + x[1] * x[1])
                          + (x[2] * x[2] + x[3] * x[3]))
                         + ((x[4] * x[4] + x[5] * x[5])
                            + (x[6] * x[6] + x[7] * x[7])))
                    mx = jnp.maximum(
                        jnp.maximum(jnp.maximum(x[0], x[1]),
                                    jnp.maximum(x[2], x[3])),
                        jnp.maximum(jnp.maximum(x[4], x[5]),
                                    jnp.maximum(x[6], x[7])))
                    mn = jnp.minimum(
                        jnp.minimum(jnp.minimum(x[0], x[1]),
                                    jnp.minimum(x[2], x[3])),
                        jnp.minimum(jnp.minimum(x[4], x[5]),
                                    jnp.minimum(x[6], x[7])))
                    return jo, s, q, mx, mn

                @pl.when(fast & ext)
                def _():
                    for j in range(nj):
                        jo, s, q, mx, mn = group_stats(j)
                        accv[pl.ds(jo, 16)] += s
                        accv[pl.ds(128 + jo, 16)] += q
                        accv[pl.ds(256 + jo, 16)] = jnp.maximum(
                            accv[pl.ds(256 + jo, 16)], mx)
                        accv[pl.ds(384 + jo, 16)] = jnp.minimum(
                            accv[pl.ds(384 + jo, 16)], mn)
                    st_s[1] = cnt + _G

                @pl.when(fast & jnp.logical_not(ext))
                def _():
                    @pl.when(cnt > 0)
                    def _():
                        fire_acc(cur, cnt)

                    for j in range(nj):
                        jo, s, q, mx, mn = group_stats(j)
                        accv[pl.ds(jo, 16)] = s
                        accv[pl.ds(128 + jo, 16)] = q
                        accv[pl.ds(256 + jo, 16)] = mx
                        accv[pl.ds(384 + jo, 16)] = mn
                    st_s[0] = f
                    st_s[1] = jnp.int32(_G)

                @pl.when(live & jnp.logical_not(uni))
                def _():
                    @pl.loop(0, _G)
                    def _(t):
                        i = t0 + t
                        sid = idsb[pl.ds(i, 16)][0]
                        cur_t = st_s[0]
                        cnt_t = st_s[1]
                        live_t = st_s[3] == 0
                        valid = live_t & (sid < hi) & (sid >= lo)
                        is_new = valid & (sid != cur_t)

                        @pl.when(live_t & (sid >= hi))
                        def _():
                            st_s[3] = jnp.int32(1)

                        @pl.when(is_new)
                        def _():
                            @pl.when(cnt_t > 0)
                            def _():
                                fire_acc(cur_t, cnt_t)

                            for j in range(nj):
                                xx = fvb[i, pl.ds(j * 16, 16)]
                                accv[pl.ds(j * 16, 16)] = xx
                                accv[pl.ds(128 + j * 16, 16)] = xx * xx
                                accv[pl.ds(256 + j * 16, 16)] = xx
                                accv[pl.ds(384 + j * 16, 16)] = xx
                            st_s[0] = sid
                            st_s[1] = jnp.int32(1)

                        @pl.when(valid & jnp.logical_not(is_new))
                        def _():
                            for j in range(nj):
                                xx = fvb[i, pl.ds(j * 16, 16)]
                                accv[pl.ds(j * 16, 16)] += xx
                                accv[pl.ds(128 + j * 16, 16)] += xx * xx
                                accv[pl.ds(256 + j * 16, 16)] = jnp.maximum(
                                    accv[pl.ds(256 + j * 16, 16)], xx)
                                accv[pl.ds(384 + j * 16, 16)] = jnp.minimum(
                                    accv[pl.ds(384 + j * 16, 16)], xx)
                            st_s[1] = cnt_t + 1

        @pl.loop(0, n // (2 * _B))
        def _(kk):
            r0 = cs + (2 * kk) * _B
            r1 = r0 + _B

            @pl.when((st_s[3] == 0) & (r0 < n))
            def _():
                @pl.when(r1 < n)
                def _():
                    issue_fetch(r1, ids1, fv1, fsem1)
                    st_s[6] = jnp.int32(1)

                wait_fetch(ids0, fv0, fsem0)
                st_s[5] = jnp.int32(0)
                process_block_grouped(r0, ids0, fv0)

            @pl.when((st_s[3] == 0) & (r1 < n))
            def _():
                @pl.when(r1 + _B < n)
                def _():
                    issue_fetch(r1 + _B, ids0, fv0, fsem0)
                    st_s[5] = jnp.int32(1)

                wait_fetch(ids1, fv1, fsem1)
                st_s[6] = jnp.int32(0)
                process_block_grouped(r1, ids1, fv1)

        @pl.when(st_s[1] > 0)
        def _():
            fire_acc(st_s[0], st_s[1])

        @pl.when(st_s[5] == 1)
        def _():
            wait_fetch(ids0, fv0, fsem0)

        @pl.when(st_s[6] == 1)
        def _():
            wait_fetch(ids1, fv1, fsem1)

        for t in range(_K):
            @pl.when(t < lax.rem(st_s[4], _K))
            def _():
                drain_one()

    return sc_kernel(feats, ids)


def _tc_finalize(acc):
    bs = 400

    def body(acc_ref, out_ref):
        a = acc_ref[...]
        sm = a[:, 0:128]
        sq = a[:, 128:256]
        mx = a[:, 256:384]
        mn = a[:, 384:512]
        cnt = a[:, 512:513]
        c1 = jnp.maximum(cnt, 1.0)
        mean = sm / c1
        var = (sq - cnt * mean * mean) / jnp.maximum(cnt - 1.0, 1.0)
        std = jnp.sqrt(jnp.clip(var, 0.0) + 1e-12)
        pos = cnt > 0.0
        out_ref[:, 0:128] = mean
        out_ref[:, 128:256] = std
        out_ref[:, 256:384] = jnp.where(pos, mx, 0.0)
        out_ref[:, 384:512] = jnp.where(pos, mn, 0.0)

    return pl.pallas_call(
        body,
        grid=(_S // bs,),
        in_specs=[pl.BlockSpec((bs, _ACC_W), lambda i: (i, 0))],
        out_specs=pl.BlockSpec((bs, 512), lambda i: (i, 0)),
        out_shape=jax.ShapeDtypeStruct((_S, 512), jnp.float32),
    )(acc)


def kernel(feats, segment_ids):
    ids = segment_ids.astype(jnp.int32)
    acc = _sc_segment_acc(feats, ids)
    return _tc_finalize(acc.reshape(_S, _ACC_W))


# speedup vs baseline: 3.3003x; 1.0098x over previous
import functools

import jax
import jax.numpy as jnp
from jax import lax
from jax.experimental import pallas as pl
from jax.experimental.pallas import tpu as pltpu
from jax.experimental.pallas import tpu_sc as plsc

_S = 10000
_NW = 32
_B = 200
_G = 4
_ACC_W = 640
_ZB = 8
_K = 8

_NEG = float("-inf")
_POS = float("inf")


def _sc_segment_acc(feats, ids):
    n, d = feats.shape
    nj = d // 16
    chunk = n // _NW
    mesh = plsc.VectorSubcoreMesh(core_axis_name="c", subcore_axis_name="s")

    @functools.partial(
        pl.kernel,
        out_type=jax.ShapeDtypeStruct((_S * _ACC_W,), jnp.float32),
        mesh=mesh,
        scratch_types=[
            pltpu.VMEM((_B + 16,), jnp.int32),
            pltpu.VMEM((_B + 16,), jnp.int32),
            pltpu.VMEM((_B, d), jnp.float32),
            pltpu.VMEM((_B, d), jnp.float32),
            pltpu.VMEM((_ACC_W,), jnp.float32),
            pltpu.VMEM((_K, _ACC_W), jnp.float32),
            pltpu.VMEM((_ZB * _ACC_W,), jnp.float32),
            pltpu.VMEM((16,), jnp.int32),
            pltpu.SMEM((8,), jnp.int32),
            pltpu.SemaphoreType.DMA,
            pltpu.SemaphoreType.DMA,
            pltpu.SemaphoreType.DMA,
        ],
    )
    def sc_kernel(feats_hbm, ids_hbm, acc_hbm, ids0, ids1, fv0, fv1,
                  accv, stg, zv, bv, st_s, fsem0, fsem1, wsem):
        w = lax.axis_index("c") * 16 + lax.axis_index("s")
        cs = w * chunk
        ce = cs + chunk

        def issue_fetch(r, idsb, fvb, fsem):
            pltpu.async_copy(
                ids_hbm.at[pl.ds(pl.multiple_of(r, 8), _B)],
                idsb.at[pl.ds(0, _B)], fsem)
            pltpu.async_copy(
                feats_hbm.at[pl.ds(pl.multiple_of(r, 8), _B)], fvb, fsem)

        def wait_fetch(idsb, fvb, fsem):
            pltpu.make_async_copy(
                ids_hbm.at[pl.ds(0, _B)], idsb.at[pl.ds(0, _B)], fsem).wait()
            pltpu.make_async_copy(
                feats_hbm.at[pl.ds(0, _B)], fvb, fsem).wait()

        issue_fetch(cs, ids0, fv0, fsem0)
        st_s[5] = jnp.int32(1)
        st_s[6] = jnp.int32(0)

        @pl.loop(0, _ZB * _ACC_W, step=16)
        def _(i):
            zv[pl.ds(i, 16)] = jnp.zeros((16,), jnp.float32)

        @pl.loop(0, _ACC_W, step=16)
        def _(i):
            accv[pl.ds(i, 16)] = jnp.zeros((16,), jnp.float32)

        pltpu.sync_copy(
            ids_hbm.at[pl.ds(pl.multiple_of(jnp.maximum(cs - 16, 0), 8), 16)],
            bv)
        prev = jnp.where(w > 0, bv[pl.ds(0, 16)][15], -1)
        pltpu.sync_copy(ids_hbm.at[pl.ds(pl.multiple_of(ce - 16, 8), 16)], bv)
        hi = jnp.where(w < _NW - 1, bv[pl.ds(0, 16)][15] + 1, _S)
        lo = prev + 1

        lo8 = ((lo + _ZB - 1) // _ZB) * _ZB
        hi8 = (hi // _ZB) * _ZB

        def zero_row(z):
            pltpu.sync_copy(
                zv.at[pl.ds(0, _ACC_W)],
                acc_hbm.at[pl.ds(pl.multiple_of(z * _ACC_W, 8), _ACC_W)])

        for t in range(_ZB - 1):
            @pl.when(lo + t < jnp.minimum(hi, lo8))
            def _(t=t):
                zero_row(lo + t)

        for t in range(_ZB - 1):
            @pl.when((hi8 + t >= lo) & (hi8 + t < hi))
            def _(t=t):
                zero_row(hi8 + t)

        @pl.loop(0, _S // _ZB)
        def _(b):
            z = b * _ZB

            @pl.when((z >= lo8) & (z < hi8))
            def _():
                pltpu.sync_copy(
                    zv,
                    acc_hbm.at[pl.ds(pl.multiple_of(z * _ACC_W, 8),
                                     _ZB * _ACC_W)])

        def drain_one():
            pltpu.make_async_copy(
                stg.at[0], acc_hbm.at[pl.ds(0, _ACC_W)], wsem).wait()

        def fire_acc(cur, cnt):
            m = st_s[4]
            sl = lax.rem(m, _K)
            accv[pl.ds(512, 16)] = jnp.full((16,), cnt.astype(jnp.float32))
            for q in range(_ACC_W // 16):
                stg[sl, pl.ds(q * 16, 16)] = accv[pl.ds(q * 16, 16)]
            pltpu.async_copy(
                stg.at[sl],
                acc_hbm.at[pl.ds(pl.multiple_of(cur * _ACC_W, 8), _ACC_W)],
                wsem)
            st_s[4] = m + 1

            @pl.when(lax.rem(m + 1, _K) == 0)
            def _():
                for _q in range(_K):
                    drain_one()

        st_s[0] = jnp.int32(-1)
        st_s[1] = jnp.int32(0)
        st_s[3] = jnp.int32(0)
        st_s[4] = jnp.int32(0)

        def process_block_grouped(r0, idsb, fvb):
            @pl.loop(0, _B // _G)
            def _(g):
                t0 = g * _G
                idv = idsb[pl.ds(t0, 16)]
                f = idv[0]
                last = idv[_G - 1]
                cur = st_s[0]
                cnt = st_s[1]
                live = st_s[3] == 0
                uni = f == last

                @pl.when(live & uni & (f >= hi))
                def _():
                    st_s[3] = jnp.int32(1)

                fast = live & uni & (f >= lo) & (f < hi)
                ext = f == cur

                def _tree(vals, op):
                    while len(vals) > 1:
                        vals = [op(vals[2 * i], vals[2 * i + 1])
                                for i in range(len(vals) // 2)]
                    return vals[0]

                def group_stats(j):
                    jo = j * 16
                    x = [fvb[t0 + t, pl.ds(jo, 16)] for t in range(_G)]
                    s = _tree(list(x), lambda a, b: a + b)
                    q = _tree([v * v for v in x], lambda a, b: a + b)
                    mx = _tree(list(x), jnp.maximum)
                    mn = _tree(list(x), jnp.minimum)
                    return jo, s, q, mx, mn

                @pl.when(fast & ext)
                def _():
                    for j in range(nj):
                        jo, s, q, mx, mn = group_stats(j)
                        accv[pl.ds(jo, 16)] += s
                        accv[pl.ds(128 + jo, 16)] += q
                        accv[pl.ds(256 + jo, 16)] = jnp.maximum(
                            accv[pl.ds(256 + jo, 16)], mx)
                        accv[pl.ds(384 + jo, 16)] = jnp.minimum(
                            accv[pl.ds(384 + jo, 16)], mn)
                    st_s[1] = cnt + _G

                @pl.when(fast & jnp.logical_not(ext))
                def _():
                    @pl.when(cnt > 0)
                    def _():
                        fire_acc(cur, cnt)

                    for j in range(nj):
                        jo, s, q, mx, mn = group_stats(j)
                        accv[pl.ds(jo, 16)] = s
                        accv[pl.ds(128 + jo, 16)] = q
                        accv[pl.ds(256 + jo, 16)] = mx
                        accv[pl.ds(384 + jo, 16)] = mn
                    st_s[0] = f
                    st_s[1] = jnp.int32(_G)

                @pl.when(live & jnp.logical_not(uni))
                def _():
                    @pl.loop(0, _G)
                    def _(t):
                        i = t0 + t
                        sid = idsb[pl.ds(i, 16)][0]
                        cur_t = st_s[0]
                        cnt_t = st_s[1]
                        live_t = st_s[3] == 0
                        valid = live_t & (sid < hi) & (sid >= lo)
                        is_new = valid & (sid != cur_t)

                        @pl.when(live_t & (sid >= hi))
                        def _():
                            st_s[3] = jnp.int32(1)

                        @pl.when(is_new)
                        def _():
                            @pl.when(cnt_t > 0)
                            def _():
                                fire_acc(cur_t, cnt_t)

                            for j in range(nj):
                                xx = fvb[i, pl.ds(j * 16, 16)]
                                accv[pl.ds(j * 16, 16)] = xx
                                accv[pl.ds(128 + j * 16, 16)] = xx * xx
                                accv[pl.ds(256 + j * 16, 16)] = xx
                                accv[pl.ds(384 + j * 16, 16)] = xx
                            st_s[0] = sid
                            st_s[1] = jnp.int32(1)

                        @pl.when(valid & jnp.logical_not(is_new))
                        def _():
                            for j in range(nj):
                                xx = fvb[i, pl.ds(j * 16, 16)]
                                accv[pl.ds(j * 16, 16)] += xx
                                accv[pl.ds(128 + j * 16, 16)] += xx * xx
                                accv[pl.ds(256 + j * 16, 16)] = jnp.maximum(
                                    accv[pl.ds(256 + j * 16, 16)], xx)
                                accv[pl.ds(384 + j * 16, 16)] = jnp.minimum(
                                    accv[pl.ds(384 + j * 16, 16)], xx)
                            st_s[1] = cnt_t + 1

        @pl.loop(0, n // (2 * _B))
        def _(kk):
            r0 = cs + (2 * kk) * _B
            r1 = r0 + _B

            @pl.when((st_s[3] == 0) & (r0 < n))
            def _():
                @pl.when(r1 < n)
                def _():
                    issue_fetch(r1, ids1, fv1, fsem1)
                    st_s[6] = jnp.int32(1)

                wait_fetch(ids0, fv0, fsem0)
                st_s[5] = jnp.int32(0)
                process_block_grouped(r0, ids0, fv0)

            @pl.when((st_s[3] == 0) & (r1 < n))
            def _():
                @pl.when(r1 + _B < n)
                def _():
                    issue_fetch(r1 + _B, ids0, fv0, fsem0)
                    st_s[5] = jnp.int32(1)

                wait_fetch(ids1, fv1, fsem1)
                st_s[6] = jnp.int32(0)
                process_block_grouped(r1, ids1, fv1)

        @pl.when(st_s[1] > 0)
        def _():
            fire_acc(st_s[0], st_s[1])

        @pl.when(st_s[5] == 1)
        def _():
            wait_fetch(ids0, fv0, fsem0)

        @pl.when(st_s[6] == 1)
        def _():
            wait_fetch(ids1, fv1, fsem1)

        for t in range(_K):
            @pl.when(t < lax.rem(st_s[4], _K))
            def _():
                drain_one()

    return sc_kernel(feats, ids)


def _tc_finalize(acc):
    bs = 400

    def body(acc_ref, out_ref):
        a = acc_ref[...]
        sm = a[:, 0:128]
        sq = a[:, 128:256]
        mx = a[:, 256:384]
        mn = a[:, 384:512]
        cnt = a[:, 512:513]
        c1 = jnp.maximum(cnt, 1.0)
        mean = sm / c1
        var = (sq - cnt * mean * mean) / jnp.maximum(cnt - 1.0, 1.0)
        std = jnp.sqrt(jnp.clip(var, 0.0) + 1e-12)
        pos = cnt > 0.0
        out_ref[:, 0:128] = mean
        out_ref[:, 128:256] = std
        out_ref[:, 256:384] = jnp.where(pos, mx, 0.0)
        out_ref[:, 384:512] = jnp.where(pos, mn, 0.0)

    return pl.pallas_call(
        body,
        grid=(_S // bs,),
        in_specs=[pl.BlockSpec((bs, _ACC_W), lambda i: (i, 0))],
        out_specs=pl.BlockSpec((bs, 512), lambda i: (i, 0)),
        out_shape=jax.ShapeDtypeStruct((_S, 512), jnp.float32),
    )(acc)


def kernel(feats, segment_ids):
    ids = segment_ids.astype(jnp.int32)
    acc = _sc_segment_acc(feats, ids)
    return _tc_finalize(acc.reshape(_S, _ACC_W))


# speedup vs baseline: 3.6259x; 1.0986x over previous
import functools

import jax
import jax.numpy as jnp
from jax import lax
from jax.experimental import pallas as pl
from jax.experimental.pallas import tpu as pltpu
from jax.experimental.pallas import tpu_sc as plsc

_S = 10000
_NW = 32
_B = 200
_G = 4
_ACC_W = 640
_ZB = 8
_K = 8

_NEG = float("-inf")
_POS = float("inf")


def _sc_segment_acc(feats, ids):
    n, d = feats.shape
    nj = d // 16
    chunk = n // _NW
    mesh = plsc.VectorSubcoreMesh(core_axis_name="c", subcore_axis_name="s")

    @functools.partial(
        pl.kernel,
        out_type=jax.ShapeDtypeStruct((_S * _ACC_W,), jnp.float32),
        mesh=mesh,
        scratch_types=[
            pltpu.VMEM((_B + 16,), jnp.int32),
            pltpu.VMEM((_B + 16,), jnp.int32),
            pltpu.VMEM((_B, d), jnp.float32),
            pltpu.VMEM((_B, d), jnp.float32),
            pltpu.VMEM((_K, _ACC_W), jnp.float32),
            pltpu.VMEM((_ZB * _ACC_W,), jnp.float32),
            pltpu.VMEM((16,), jnp.int32),
            pltpu.SMEM((8,), jnp.int32),
            pltpu.SemaphoreType.DMA,
            pltpu.SemaphoreType.DMA,
            pltpu.SemaphoreType.DMA,
            pltpu.SemaphoreType.DMA,
        ],
    )
    def sc_kernel(feats_hbm, ids_hbm, acc_hbm, ids0, ids1, fv0, fv1,
                  stg, zv, bv, st_s, fsem0, fsem1, wsem, zsem):
        w = lax.axis_index("c") * 16 + lax.axis_index("s")
        cs = w * chunk
        ce = cs + chunk

        def issue_fetch(r, idsb, fvb, fsem):
            pltpu.async_copy(
                ids_hbm.at[pl.ds(pl.multiple_of(r, 8), _B)],
                idsb.at[pl.ds(0, _B)], fsem)
            pltpu.async_copy(
                feats_hbm.at[pl.ds(pl.multiple_of(r, 8), _B)], fvb, fsem)

        def wait_fetch(idsb, fvb, fsem):
            pltpu.make_async_copy(
                ids_hbm.at[pl.ds(0, _B)], idsb.at[pl.ds(0, _B)], fsem).wait()
            pltpu.make_async_copy(
                feats_hbm.at[pl.ds(0, _B)], fvb, fsem).wait()

        issue_fetch(cs, ids0, fv0, fsem0)
        st_s[5] = jnp.int32(1)
        st_s[6] = jnp.int32(0)

        @pl.loop(0, _ZB * _ACC_W, step=16)
        def _(i):
            zv[pl.ds(i, 16)] = jnp.zeros((16,), jnp.float32)

        pltpu.sync_copy(
            ids_hbm.at[pl.ds(pl.multiple_of(jnp.maximum(cs - 16, 0), 8), 16)],
            bv)
        prev = jnp.where(w > 0, bv[pl.ds(0, 16)][15], -1)
        pltpu.sync_copy(ids_hbm.at[pl.ds(pl.multiple_of(ce - 16, 8), 16)], bv)
        hi = jnp.where(w < _NW - 1, bv[pl.ds(0, 16)][15] + 1, _S)
        lo = prev + 1

        lo8 = ((lo + _ZB - 1) // _ZB) * _ZB
        hi8 = (hi // _ZB) * _ZB

        def zero_row(z):
            pltpu.async_copy(
                zv.at[pl.ds(0, _ACC_W)],
                acc_hbm.at[pl.ds(pl.multiple_of(z * _ACC_W, 8), _ACC_W)],
                zsem)

        def zero_row_wait():
            pltpu.make_async_copy(
                zv.at[pl.ds(0, _ACC_W)],
                acc_hbm.at[pl.ds(0, _ACC_W)], zsem).wait()

        nblk = jnp.maximum(hi8 - lo8, 0) // _ZB

        for t in range(_ZB - 1):
            @pl.when(lo + t < jnp.minimum(hi, lo8))
            def _(t=t):
                zero_row(lo + t)

        for t in range(_ZB - 1):
            @pl.when((hi8 + t >= lo) & (hi8 + t < hi))
            def _(t=t):
                zero_row(hi8 + t)

        @pl.loop(0, nblk)
        def _(b):
            pltpu.async_copy(
                zv,
                acc_hbm.at[pl.ds(pl.multiple_of((lo8 + b * _ZB) * _ACC_W, 8),
                                 _ZB * _ACC_W)], zsem)

        for t in range(_ZB - 1):
            @pl.when(lo + t < jnp.minimum(hi, lo8))
            def _(t=t):
                zero_row_wait()

        for t in range(_ZB - 1):
            @pl.when((hi8 + t >= lo) & (hi8 + t < hi))
            def _(t=t):
                zero_row_wait()

        @pl.loop(0, nblk)
        def _(b):
            pltpu.make_async_copy(
                zv, acc_hbm.at[pl.ds(0, _ZB * _ACC_W)], zsem).wait()

        def drain_one():
            pltpu.make_async_copy(
                stg.at[0], acc_hbm.at[pl.ds(0, _ACC_W)], wsem).wait()

        def fire_acc(cur, cnt):
            m = st_s[4]
            sl = st_s[2]
            stg[sl, pl.ds(512, 16)] = jnp.full((16,), cnt.astype(jnp.float32))
            pltpu.async_copy(
                stg.at[sl],
                acc_hbm.at[pl.ds(pl.multiple_of(cur * _ACC_W, 8), _ACC_W)],
                wsem)
            st_s[4] = m + 1
            st_s[2] = lax.rem(m + 1, _K)

            @pl.when(lax.rem(m + 1, _K) == 0)
            def _():
                for _q in range(_K):
                    drain_one()

        st_s[0] = jnp.int32(-1)
        st_s[1] = jnp.int32(0)
        st_s[2] = jnp.int32(0)
        st_s[3] = jnp.int32(0)
        st_s[4] = jnp.int32(0)

        def process_block_grouped(r0, idsb, fvb):
            @pl.loop(0, _B // _G)
            def _(g):
                t0 = g * _G
                idv = idsb[pl.ds(t0, 16)]
                f = idv[0]
                last = idv[_G - 1]
                cur = st_s[0]
                cnt = st_s[1]
                live = st_s[3] == 0
                uni = f == last

                @pl.when(live & uni & (f >= hi))
                def _():
                    st_s[3] = jnp.int32(1)

                fast = live & uni & (f >= lo) & (f < hi)
                ext = f == cur

                def _tree(vals, op):
                    while len(vals) > 1:
                        vals = [op(vals[2 * i], vals[2 * i + 1])
                                for i in range(len(vals) // 2)]
                    return vals[0]

                def group_stats(j):
                    jo = j * 16
                    x = [fvb[t0 + t, pl.ds(jo, 16)] for t in range(_G)]
                    s = _tree(list(x), lambda a, b: a + b)
                    q = _tree([v * v for v in x], lambda a, b: a + b)
                    mx = _tree(list(x), jnp.maximum)
                    mn = _tree(list(x), jnp.minimum)
                    return jo, s, q, mx, mn

                @pl.when(fast & ext)
                def _():
                    slv = st_s[2]
                    for j in range(nj):
                        jo, s, q, mx, mn = group_stats(j)
                        stg[slv, pl.ds(jo, 16)] += s
                        stg[slv, pl.ds(128 + jo, 16)] += q
                        stg[slv, pl.ds(256 + jo, 16)] = jnp.maximum(
                            stg[slv, pl.ds(256 + jo, 16)], mx)
                        stg[slv, pl.ds(384 + jo, 16)] = jnp.minimum(
                            stg[slv, pl.ds(384 + jo, 16)], mn)
                    st_s[1] = cnt + _G

                @pl.when(fast & jnp.logical_not(ext))
                def _():
                    @pl.when(cnt > 0)
                    def _():
                        fire_acc(cur, cnt)

                    slv = st_s[2]
                    for j in range(nj):
                        jo, s, q, mx, mn = group_stats(j)
                        stg[slv, pl.ds(jo, 16)] = s
                        stg[slv, pl.ds(128 + jo, 16)] = q
                        stg[slv, pl.ds(256 + jo, 16)] = mx
                        stg[slv, pl.ds(384 + jo, 16)] = mn
                    st_s[0] = f
                    st_s[1] = jnp.int32(_G)

                @pl.when(live & jnp.logical_not(uni))
                def _():
                    @pl.loop(0, _G)
                    def _(t):
                        i = t0 + t
                        sid = idsb[pl.ds(i, 16)][0]
                        cur_t = st_s[0]
                        cnt_t = st_s[1]
                        live_t = st_s[3] == 0
                        valid = live_t & (sid < hi) & (sid >= lo)
                        is_new = valid & (sid != cur_t)

                        @pl.when(live_t & (sid >= hi))
                        def _():
                            st_s[3] = jnp.int32(1)

                        @pl.when(is_new)
                        def _():
                            @pl.when(cnt_t > 0)
                            def _():
                                fire_acc(cur_t, cnt_t)

                            slv = st_s[2]
                            for j in range(nj):
                                xx = fvb[i, pl.ds(j * 16, 16)]
                                stg[slv, pl.ds(j * 16, 16)] = xx
                                stg[slv, pl.ds(128 + j * 16, 16)] = xx * xx
                                stg[slv, pl.ds(256 + j * 16, 16)] = xx
                                stg[slv, pl.ds(384 + j * 16, 16)] = xx
                            st_s[0] = sid
                            st_s[1] = jnp.int32(1)

                        @pl.when(valid & jnp.logical_not(is_new))
                        def _():
                            slv = st_s[2]
                            for j in range(nj):
                                xx = fvb[i, pl.ds(j * 16, 16)]
                                stg[slv, pl.ds(j * 16, 16)] += xx
                                stg[slv, pl.ds(128 + j * 16, 16)] += xx * xx
                                stg[slv, pl.ds(256 + j * 16, 16)] = jnp.maximum(
                                    stg[slv, pl.ds(256 + j * 16, 16)], xx)
                                stg[slv, pl.ds(384 + j * 16, 16)] = jnp.minimum(
                                    stg[slv, pl.ds(384 + j * 16, 16)], xx)
                            st_s[1] = cnt_t + 1

        @pl.loop(0, n // (2 * _B))
        def _(kk):
            r0 = cs + (2 * kk) * _B
            r1 = r0 + _B

            @pl.when((st_s[3] == 0) & (r0 < n))
            def _():
                @pl.when(r1 < n)
                def _():
                    issue_fetch(r1, ids1, fv1, fsem1)
                    st_s[6] = jnp.int32(1)

                wait_fetch(ids0, fv0, fsem0)
                st_s[5] = jnp.int32(0)
                process_block_grouped(r0, ids0, fv0)

            @pl.when((st_s[3] == 0) & (r1 < n))
            def _():
                @pl.when(r1 + _B < n)
                def _():
                    issue_fetch(r1 + _B, ids0, fv0, fsem0)
                    st_s[5] = jnp.int32(1)

                wait_fetch(ids1, fv1, fsem1)
                st_s[6] = jnp.int32(0)
                process_block_grouped(r1, ids1, fv1)

        @pl.when(st_s[1] > 0)
        def _():
            fire_acc(st_s[0], st_s[1])

        @pl.when(st_s[5] == 1)
        def _():
            wait_fetch(ids0, fv0, fsem0)

        @pl.when(st_s[6] == 1)
        def _():
            wait_fetch(ids1, fv1, fsem1)

        for t in range(_K):
            @pl.when(t < lax.rem(st_s[4], _K))
            def _():
                drain_one()

    return sc_kernel(feats, ids)


def _tc_finalize(acc):
    bs = 400

    def body(acc_ref, out_ref):
        a = acc_ref[...]
        sm = a[:, 0:128]
        sq = a[:, 128:256]
        mx = a[:, 256:384]
        mn = a[:, 384:512]
        cnt = a[:, 512:513]
        c1 = jnp.maximum(cnt, 1.0)
        mean = sm / c1
        var = (sq - cnt * mean * mean) / jnp.maximum(cnt - 1.0, 1.0)
        std = jnp.sqrt(jnp.clip(var, 0.0) + 1e-12)
        pos = cnt > 0.0
        out_ref[:, 0:128] = mean
        out_ref[:, 128:256] = std
        out_ref[:, 256:384] = jnp.where(pos, mx, 0.0)
        out_ref[:, 384:512] = jnp.where(pos, mn, 0.0)

    return pl.pallas_call(
        body,
        grid=(_S // bs,),
        in_specs=[pl.BlockSpec((bs, _ACC_W), lambda i: (i, 0))],
        out_specs=pl.BlockSpec((bs, 512), lambda i: (i, 0)),
        out_shape=jax.ShapeDtypeStruct((_S, 512), jnp.float32),
    )(acc)


def kernel(feats, segment_ids):
    ids = segment_ids.astype(jnp.int32)
    acc = _sc_segment_acc(feats, ids)
    return _tc_finalize(acc.reshape(_S, _ACC_W))
